# Initial kernel scaffold; baseline (speedup 1.0000x reference)
import jax
import jax.numpy as jnp
from jax.experimental import pallas as pl


def kernel(z, W):
    raise NotImplementedError("write your pallas kernel here")


# speedup vs baseline: 2.0764x; 2.0764x over previous
import jax
import jax.numpy as jnp
from jax.experimental import pallas as pl
from jax.experimental.pallas import tpu as pltpu

_NE = 1024
_ED = 256
_BETA = 0.25
_BLK = 1024


def _vq_body(z_ref, w_ref, idx_ref, zq_ref, loss_ref, ppl_ref, cnt_ref):
    i = pl.program_id(0)
    nsteps = pl.num_programs(0)
    z = z_ref[...]
    w = w_ref[...]

    zw = jax.lax.dot_general(z, w, (((1,), (1,)), ((), ())),
                             preferred_element_type=jnp.float32,
                             precision=jax.lax.Precision.DEFAULT)
    z2 = jnp.sum(z * z, axis=1, keepdims=True)
    w2 = jnp.sum(w * w, axis=1)
    d = (z2 + w2[None, :]) - 2.0 * zw

    dmin = jnp.min(d, axis=1, keepdims=True)
    lanes = jax.lax.broadcasted_iota(jnp.int32, d.shape, 1)
    idx = jnp.min(jnp.where(d == dmin, lanes, _NE), axis=1).astype(jnp.int32)
    idx_ref[0, 0, :] = idx

    oh = lanes == idx[:, None]
    zq = jax.lax.dot_general(oh.astype(jnp.bfloat16), w.astype(jnp.bfloat16),
                             (((1,), (0,)), ((), ())),
                             preferred_element_type=jnp.float32)
    zq_ref[...] = zq

    sq = jnp.sum((zq - z) ** 2)
    cnt_step = jnp.sum(oh.astype(jnp.float32), axis=0, keepdims=True)

    @pl.when(i == 0)
    def _init():
        loss_ref[0, 0] = 0.0
        cnt_ref[...] = jnp.zeros_like(cnt_ref)

    loss_ref[0, 0] += sq
    cnt_ref[...] += cnt_step

    @pl.when(i == nsteps - 1)
    def _finish():
        total = jnp.float32(nsteps * _BLK)
        e_mean = cnt_ref[...] / total
        ent = jnp.sum(e_mean * jnp.log(e_mean + 1e-10))
        ppl_ref[0, 0] = jnp.exp(-ent)
        loss_ref[0, 0] = (1.0 + _BETA) * loss_ref[0, 0] / (total * _ED)


def kernel(z, W):
    zf = z.reshape(-1, _ED)
    m = zf.shape[0]
    nb = m // _BLK
    idx3, zq, loss, ppl = pl.pallas_call(
        _vq_body,
        grid=(nb,),
        in_specs=[
            pl.BlockSpec((_BLK, _ED), lambda i: (i, 0)),
            pl.BlockSpec((_NE, _ED), lambda i: (0, 0)),
        ],
        out_specs=[
            pl.BlockSpec((1, 1, _BLK), lambda i: (i, 0, 0)),
            pl.BlockSpec((_BLK, _ED), lambda i: (i, 0)),
            pl.BlockSpec(memory_space=pltpu.SMEM),
            pl.BlockSpec(memory_space=pltpu.SMEM),
        ],
        out_shape=[
            jax.ShapeDtypeStruct((nb, 1, _BLK), jnp.int32),
            jax.ShapeDtypeStruct((m, _ED), jnp.float32),
            jax.ShapeDtypeStruct((1, 1), jnp.float32),
            jax.ShapeDtypeStruct((1, 1), jnp.float32),
        ],
        scratch_shapes=[pltpu.VMEM((1, _NE), jnp.float32)],
    )(zf, W)
    return (loss[0, 0], zq.reshape(z.shape), idx3.reshape(m), ppl[0, 0])


# speedup vs baseline: 2.1283x; 1.0250x over previous
import jax
import jax.numpy as jnp
from jax.experimental import pallas as pl
from jax.experimental.pallas import tpu as pltpu

_NE = 1024
_ED = 256
_BETA = 0.25
_BLK = 1024


def _vq_body(z_ref, w_ref, idx_ref, zq_ref, loss_ref, ppl_ref, cnt_ref):
    i = pl.program_id(0)
    nsteps = pl.num_programs(0)
    z = z_ref[...]
    w = w_ref[...]

    zw = jax.lax.dot_general(z, w, (((1,), (1,)), ((), ())),
                             preferred_element_type=jnp.float32,
                             precision=jax.lax.Precision.DEFAULT)
    z2 = jnp.sum(z * z, axis=1, keepdims=True)
    w2 = jnp.sum(w * w, axis=1)
    d = (z2 + w2[None, :]) - 2.0 * zw

    idx = jnp.argmin(d, axis=1).astype(jnp.int32)
    idx_ref[0, 0, :] = idx
    lanes = jax.lax.broadcasted_iota(jnp.int32, d.shape, 1)

    oh = lanes == idx[:, None]
    zq = jax.lax.dot_general(oh.astype(jnp.bfloat16), w.astype(jnp.bfloat16),
                             (((1,), (0,)), ((), ())),
                             preferred_element_type=jnp.float32)
    zq_ref[...] = zq

    sq = jnp.sum((zq - z) ** 2)
    cnt_step = jnp.sum(oh.astype(jnp.float32), axis=0, keepdims=True)

    @pl.when(i == 0)
    def _init():
        loss_ref[0, 0] = 0.0
        cnt_ref[...] = jnp.zeros_like(cnt_ref)

    loss_ref[0, 0] += sq
    cnt_ref[...] += cnt_step

    @pl.when(i == nsteps - 1)
    def _finish():
        total = jnp.float32(nsteps * _BLK)
        e_mean = cnt_ref[...] / total
        ent = jnp.sum(e_mean * jnp.log(e_mean + 1e-10))
        ppl_ref[0, 0] = jnp.exp(-ent)
        loss_ref[0, 0] = (1.0 + _BETA) * loss_ref[0, 0] / (total * _ED)


def kernel(z, W):
    zf = z.reshape(-1, _ED)
    m = zf.shape[0]
    nb = m // _BLK
    idx3, zq, loss, ppl = pl.pallas_call(
        _vq_body,
        grid=(nb,),
        in_specs=[
            pl.BlockSpec((_BLK, _ED), lambda i: (i, 0)),
            pl.BlockSpec((_NE, _ED), lambda i: (0, 0)),
        ],
        out_specs=[
            pl.BlockSpec((1, 1, _BLK), lambda i: (i, 0, 0)),
            pl.BlockSpec((_BLK, _ED), lambda i: (i, 0)),
            pl.BlockSpec(memory_space=pltpu.SMEM),
            pl.BlockSpec(memory_space=pltpu.SMEM),
        ],
        out_shape=[
            jax.ShapeDtypeStruct((nb, 1, _BLK), jnp.int32),
            jax.ShapeDtypeStruct((m, _ED), jnp.float32),
            jax.ShapeDtypeStruct((1, 1), jnp.float32),
            jax.ShapeDtypeStruct((1, 1), jnp.float32),
        ],
        scratch_shapes=[pltpu.VMEM((1, _NE), jnp.float32)],
    )(zf, W)
    return (loss[0, 0], zq.reshape(z.shape), idx3.reshape(m), ppl[0, 0])


# speedup vs baseline: 2.3258x; 1.0928x over previous
import jax
import jax.numpy as jnp
from jax.experimental import pallas as pl
from jax.experimental.pallas import tpu as pltpu

_NE = 1024
_ED = 256
_BETA = 0.25
_BLK = 4096


def _vq_body(z_ref, w_ref, idx_ref, zq_ref, loss_ref, ppl_ref, cnt_ref):
    i = pl.program_id(0)
    nsteps = pl.num_programs(0)
    z = z_ref[...]
    w = w_ref[...]

    zw = jax.lax.dot_general(z, w, (((1,), (1,)), ((), ())),
                             preferred_element_type=jnp.float32,
                             precision=jax.lax.Precision.DEFAULT)
    z2 = jnp.sum(z * z, axis=1, keepdims=True)
    w2 = jnp.sum(w * w, axis=1)
    d = (z2 + w2[None, :]) - 2.0 * zw

    dmin = jnp.min(d, axis=1, keepdims=True)
    lanes = jax.lax.broadcasted_iota(jnp.int32, d.shape, 1)
    idx = jnp.min(jnp.where(d == dmin, lanes, _NE), axis=1).astype(jnp.int32)
    idx_ref[0, 0, :] = idx

    oh = lanes == idx[:, None]
    zq = jax.lax.dot_general(oh.astype(jnp.bfloat16), w.astype(jnp.bfloat16),
                             (((1,), (0,)), ((), ())),
                             preferred_element_type=jnp.float32)
    zq_ref[...] = zq

    sq = jnp.sum((zq - z) ** 2)
    cnt_step = jnp.sum(oh.astype(jnp.float32), axis=0, keepdims=True)

    @pl.when(i == 0)
    def _init():
        loss_ref[0, 0] = 0.0
        cnt_ref[...] = jnp.zeros_like(cnt_ref)

    loss_ref[0, 0] += sq
    cnt_ref[...] += cnt_step

    @pl.when(i == nsteps - 1)
    def _finish():
        total = jnp.float32(nsteps * _BLK)
        e_mean = cnt_ref[...] / total
        ent = jnp.sum(e_mean * jnp.log(e_mean + 1e-10))
        ppl_ref[0, 0] = jnp.exp(-ent)
        loss_ref[0, 0] = (1.0 + _BETA) * loss_ref[0, 0] / (total * _ED)


def kernel(z, W):
    zf = z.reshape(-1, _ED)
    m = zf.shape[0]
    nb = m // _BLK
    idx3, zq, loss, ppl = pl.pallas_call(
        _vq_body,
        grid=(nb,),
        in_specs=[
            pl.BlockSpec((_BLK, _ED), lambda i: (i, 0)),
            pl.BlockSpec((_NE, _ED), lambda i: (0, 0)),
        ],
        out_specs=[
            pl.BlockSpec((1, 1, _BLK), lambda i: (i, 0, 0)),
            pl.BlockSpec((_BLK, _ED), lambda i: (i, 0)),
            pl.BlockSpec(memory_space=pltpu.SMEM),
            pl.BlockSpec(memory_space=pltpu.SMEM),
        ],
        out_shape=[
            jax.ShapeDtypeStruct((nb, 1, _BLK), jnp.int32),
            jax.ShapeDtypeStruct((m, _ED), jnp.float32),
            jax.ShapeDtypeStruct((1, 1), jnp.float32),
            jax.ShapeDtypeStruct((1, 1), jnp.float32),
        ],
        scratch_shapes=[pltpu.VMEM((1, _NE), jnp.float32)],
    )(zf, W)
    return (loss[0, 0], zq.reshape(z.shape), idx3.reshape(m), ppl[0, 0])


# speedup vs baseline: 2.6519x; 1.1402x over previous
import numpy as np

import jax
import jax.numpy as jnp
from jax.experimental import pallas as pl
from jax.experimental.pallas import tpu as pltpu

_NE = 1024
_ED = 256
_BETA = 0.25
_BLK = 4096
_CHUNK = 16
_NCH = _NE // _CHUNK


def _consts():
    bits = np.zeros((_NE, _NCH), np.float32)
    for l in range(_NE):
        bits[l, l // _CHUNK] = 2.0 ** (15 - l % _CHUNK)
    ones = np.ones((1, _BLK), np.float32)
    return jnp.asarray(bits, jnp.bfloat16), jnp.asarray(ones, jnp.bfloat16)


def _vq_body(z_ref, w_ref, bits_ref, ones_ref,
             idx_ref, zq_ref, loss_ref, ppl_ref, cnt_ref):
    i = pl.program_id(0)
    nsteps = pl.num_programs(0)
    z = z_ref[...]
    w = w_ref[...]

    zw = jax.lax.dot_general(z, w, (((1,), (1,)), ((), ())),
                             preferred_element_type=jnp.float32,
                             precision=jax.lax.Precision.DEFAULT)
    z2 = jnp.sum(z * z, axis=1, keepdims=True)
    w2 = jnp.sum(w * w, axis=1)
    d = (z2 + w2[None, :]) - 2.0 * zw

    dmin = jnp.min(d, axis=1, keepdims=True)
    mb = (d == dmin).astype(jnp.bfloat16)
    extt = jax.lax.dot_general(bits_ref[...], mb, (((0,), (1,)), ((), ())),
                               preferred_element_type=jnp.float32)
    ebits = jax.lax.bitcast_convert_type(extt, jnp.int32)
    lic = 15 - ((ebits >> 23) - 127)
    kvec = jax.lax.broadcasted_iota(jnp.int32, extt.shape, 0)
    cand = jnp.where(extt > 0.0, kvec * 16 + lic, _NE)
    idx_row = jnp.min(cand, axis=0).astype(jnp.int32)
    idx_ref[0, 0, :] = idx_row

    subl = jax.lax.broadcasted_iota(jnp.int32, (_NE, _BLK), 0)
    oht = (subl == idx_row[None, :]).astype(jnp.bfloat16)
    zq = jax.lax.dot_general(oht, w.astype(jnp.bfloat16),
                             (((0,), (0,)), ((), ())),
                             preferred_element_type=jnp.float32)
    zq_ref[...] = zq

    dsum = jnp.sum(dmin)
    cnt_step = jax.lax.dot_general(ones_ref[...], oht, (((1,), (1,)), ((), ())),
                                   preferred_element_type=jnp.float32)

    @pl.when(i == 0)
    def _init():
        loss_ref[0, 0] = 0.0
        cnt_ref[...] = jnp.zeros_like(cnt_ref)

    loss_ref[0, 0] += dsum
    cnt_ref[...] += cnt_step

    @pl.when(i == nsteps - 1)
    def _finish():
        total = jnp.float32(nsteps * _BLK)
        e_mean = cnt_ref[...] / total
        ent = jnp.sum(e_mean * jnp.log(e_mean + 1e-10))
        ppl_ref[0, 0] = jnp.exp(-ent)
        loss_ref[0, 0] = (1.0 + _BETA) * loss_ref[0, 0] / (total * _ED)


def kernel(z, W):
    zf = z.reshape(-1, _ED)
    m = zf.shape[0]
    nb = m // _BLK
    bits, ones = _consts()
    idx3, zq, loss, ppl = pl.pallas_call(
        _vq_body,
        grid=(nb,),
        in_specs=[
            pl.BlockSpec((_BLK, _ED), lambda i: (i, 0)),
            pl.BlockSpec((_NE, _ED), lambda i: (0, 0)),
            pl.BlockSpec((_NE, _NCH), lambda i: (0, 0)),
            pl.BlockSpec((1, _BLK), lambda i: (0, 0)),
        ],
        out_specs=[
            pl.BlockSpec((1, 1, _BLK), lambda i: (i, 0, 0)),
            pl.BlockSpec((_BLK, _ED), lambda i: (i, 0)),
            pl.BlockSpec(memory_space=pltpu.SMEM),
            pl.BlockSpec(memory_space=pltpu.SMEM),
        ],
        out_shape=[
            jax.ShapeDtypeStruct((nb, 1, _BLK), jnp.int32),
            jax.ShapeDtypeStruct((m, _ED), jnp.float32),
            jax.ShapeDtypeStruct((1, 1), jnp.float32),
            jax.ShapeDtypeStruct((1, 1), jnp.float32),
        ],
        scratch_shapes=[pltpu.VMEM((1, _NE), jnp.float32)],
    )(zf, W, bits, ones)
    return (loss[0, 0], zq.reshape(z.shape), idx3.reshape(m), ppl[0, 0])
